# baseline (speedup 1.0000x reference)
import functools

import jax
import jax.numpy as jnp
from jax import lax
from jax.experimental import pallas as pl
from jax.experimental.pallas import tpu as pltpu
from jax.experimental.pallas import tpu_sc as plsc

EMBEDDING_DIM = 32


def _build_sc_gather(B, D, num_cores, num_subcores, chunk):
    nw = num_cores * num_subcores
    b_per_w = B // nw
    n_chunks = b_per_w // chunk
    n_pairs = n_chunks // 2
    mesh = plsc.VectorSubcoreMesh(core_axis_name="c", subcore_axis_name="s")

    @functools.partial(
        pl.kernel,
        mesh=mesh,
        out_type=jax.ShapeDtypeStruct((B, D), jnp.float32),
        scratch_types=[
            pltpu.VMEM((2, chunk), jnp.int32),
            pltpu.VMEM((2, chunk, D), jnp.float32),
            pltpu.SemaphoreType.DMA,
            pltpu.SemaphoreType.DMA,
            pltpu.SemaphoreType.DMA,
            pltpu.SemaphoreType.DMA,
        ],
        compiler_params=pltpu.CompilerParams(use_tc_tiling_on_sc=False),
    )
    def emb(idx_hbm, table_hbm, out_hbm, idx_v, rows_v, g0, g1, s0, s1):
        wid = lax.axis_index("s") * num_cores + lax.axis_index("c")
        base = wid * b_per_w

        def idx_slice(i):
            return idx_hbm.at[pl.ds(base + i * chunk, chunk)]

        def out_slice(i):
            return out_hbm.at[pl.ds(base + i * chunk, chunk)]

        def start_gather(i, b, sem):
            pltpu.sync_copy(idx_slice(i), idx_v.at[b])
            pltpu.async_copy(table_hbm.at[idx_v.at[b]], rows_v.at[b], sem)

        def wait_gather(b, sem):
            pltpu.make_async_copy(
                table_hbm.at[idx_v.at[b]], rows_v.at[b], sem).wait()

        start_gather(0, 0, g0)

        def body(p, carry):
            i = 2 * p
            wait_gather(0, g0)

            @pl.when(p > 0)
            def _():
                pltpu.make_async_copy(rows_v.at[1], out_slice(i - 1), s1).wait()

            start_gather(i + 1, 1, g1)
            pltpu.async_copy(rows_v.at[0], out_slice(i), s0)

            wait_gather(1, g1)
            pltpu.make_async_copy(rows_v.at[0], out_slice(i), s0).wait()

            @pl.when(p < n_pairs - 1)
            def _():
                start_gather(i + 2, 0, g0)

            pltpu.async_copy(rows_v.at[1], out_slice(i + 1), s1)
            return carry

        lax.fori_loop(0, n_pairs, body, 0)
        pltpu.make_async_copy(
            rows_v.at[1], out_slice(n_chunks - 1), s1).wait()

    return emb


def kernel(x, weight):
    B = x.shape[0] * x.shape[1]
    D = weight.shape[1]
    idx = x.reshape(B).astype(jnp.int32)
    emb = _build_sc_gather(B, D, num_cores=2, num_subcores=16, chunk=1600)
    out = emb(idx, weight)
    return out.reshape(x.shape[0], x.shape[1], D)


# speedup vs baseline: 1.2132x; 1.2132x over previous
import functools

import jax
import jax.numpy as jnp
from jax import lax
from jax.experimental import pallas as pl
from jax.experimental.pallas import tpu as pltpu
from jax.experimental.pallas import tpu_sc as plsc

EMBEDDING_DIM = 32


def _build_emb(S, Bb, V, D, num_cores, num_subcores):
    nw = num_cores * num_subcores
    bw = Bb // nw
    half = bw // 2
    S_pad = (S + 7) // 8 * 8
    n_sgroups = S_pad // 8
    mesh = plsc.VectorSubcoreMesh(core_axis_name="c", subcore_axis_name="s")

    @functools.partial(
        pl.kernel,
        mesh=mesh,
        out_type=jax.ShapeDtypeStruct((S, D, Bb), jnp.float32),
        scratch_types=[
            pltpu.VMEM((8, bw), jnp.int32),
            pltpu.VMEM((half,), jnp.int32),
            pltpu.VMEM((half,), jnp.int32),
            pltpu.VMEM((2, half, 4 * D), jnp.float32),
            pltpu.VMEM((D, bw), jnp.float32),
            pltpu.SemaphoreType.DMA,
        ],
        compiler_params=pltpu.CompilerParams(
            use_tc_tiling_on_sc=True, needs_layout_passes=False),
    )
    def emb(xT, w4, outT, idx_v, j_v, cv_v, stage, outstage, gsem):
        wid = lax.axis_index("s") * num_cores + lax.axis_index("c")
        b0 = wid * bw
        iota16 = lax.iota(jnp.int32, 16)

        def sgroup(sg, carry):
            s0 = sg * 8
            pltpu.sync_copy(xT.at[pl.ds(s0, 8), pl.ds(b0, bw)], idx_v)
            ns = jnp.minimum(8, S - s0)

            def s_body(si, carry2):
                for c in range(2):
                    def grp(g, _):
                        iv = idx_v[si, pl.ds(c * half + g * 16, 16)]
                        j_v[pl.ds(g * 16, 16)] = lax.shift_right_logical(iv, 2)
                        cv_v[pl.ds(g * 16, 16)] = lax.shift_left(
                            lax.bitwise_and(iv, 3), 5)
                        return 0

                    lax.fori_loop(0, half // 16, grp, 0)
                    pltpu.async_copy(w4.at[j_v], stage.at[c], gsem).wait()

                    def cgrp(g, _):
                        rv = iota16 + g * 16
                        cv = cv_v[pl.ds(g * 16, 16)]
                        for d in range(D):
                            vals = plsc.load_gather(stage.at[c], [rv, cv + d])
                            outstage[d, pl.ds(c * half + g * 16, 16)] = vals
                        return 0

                    lax.fori_loop(0, half // 16, cgrp, 0)

                s = s0 + si
                pltpu.sync_copy(outstage, outT.at[s, :, pl.ds(b0, bw)])
                return carry2

            lax.fori_loop(0, ns, s_body, 0)
            return carry

        lax.fori_loop(0, n_sgroups, sgroup, 0)

    return emb


def kernel(x, weight):
    Bb, S = x.shape
    V, D = weight.shape
    w4 = weight.reshape(V // 4, 4 * D)
    S_pad = (S + 7) // 8 * 8
    xT = jnp.pad(x.T, ((0, S_pad - S), (0, 0)))
    emb = _build_emb(S, Bb, V, D, num_cores=2, num_subcores=16)
    outT = emb(xT, w4)
    return outT.transpose(2, 0, 1)


# speedup vs baseline: 1.4537x; 1.1983x over previous
import functools

import jax
import jax.numpy as jnp
from jax import lax
from jax.experimental import pallas as pl
from jax.experimental.pallas import tpu as pltpu
from jax.experimental.pallas import tpu_sc as plsc

EMBEDDING_DIM = 32


def _build_emb(S, Bb, V, D, num_cores, num_subcores):
    nw = num_cores * num_subcores
    bw = Bb // nw
    ntb = bw // 128
    ntd = D // 8
    S_pad = (S + 7) // 8 * 8
    assert S == 50 and (S - 2) % 6 == 0
    n_blocks = (S - 2) // 6
    mesh = plsc.VectorSubcoreMesh(core_axis_name="c", subcore_axis_name="s")

    @functools.partial(
        pl.kernel,
        mesh=mesh,
        out_type=jax.ShapeDtypeStruct((S, ntd, Bb // 128, 8, 128), jnp.float32),
        scratch_types=[
            pltpu.VMEM((3, bw), jnp.int32),
            pltpu.VMEM((3, bw, D), jnp.float32),
            pltpu.VMEM((2, ntd, ntb, 8, 128), jnp.float32),
            pltpu.SemaphoreType.DMA,
            pltpu.SemaphoreType.DMA,
            pltpu.SemaphoreType.DMA,
            pltpu.SemaphoreType.DMA,
            pltpu.SemaphoreType.DMA,
        ],
        compiler_params=pltpu.CompilerParams(
            use_tc_tiling_on_sc=False, needs_layout_passes=False),
    )
    def emb(xT, w_rows, out5, idx_v, stage, outstage, gs0, gs1, gs2, os0, os1):
        gsems = [gs0, gs1, gs2]
        osems = [os0, os1]
        wid = lax.axis_index("s") * num_cores + lax.axis_index("c")
        b0 = wid * bw
        tb0 = wid * ntb
        iota16 = lax.iota(jnp.int32, 16)

        def fire_gather(s, ph):
            g = ph % 3
            pltpu.sync_copy(xT.at[s, pl.ds(b0, bw)], idx_v.at[g])
            pltpu.async_copy(w_rows.at[idx_v.at[g]], stage.at[g], gsems[g])

        def wait_gather(ph):
            g = ph % 3
            pltpu.make_async_copy(
                w_rows.at[idx_v.at[g]], stage.at[g], gsems[g]).wait()

        def fire_out(s, ph):
            o = ph % 2
            for td in range(ntd):
                pltpu.async_copy(
                    outstage.at[o, td], out5.at[s, td, pl.ds(tb0, ntb)],
                    osems[o])

        def drain_out(s_old, ph):
            o = ph % 2
            for td in range(ntd):
                pltpu.make_async_copy(
                    outstage.at[o, td], out5.at[s_old, td, pl.ds(tb0, ntb)],
                    osems[o]).wait()

        def compact(ph):
            g, o = ph % 3, ph % 2

            def body(r, carry):
                rv = iota16 + r * 16
                tb = r // 8
                bl0 = (r % 8) * 16
                for d in range(D):
                    dv = jnp.full((16,), d, dtype=jnp.int32)
                    vals = plsc.load_gather(stage.at[g], [rv, dv])
                    outstage[o, d // 8, tb, d % 8, pl.ds(bl0, 16)] = vals
                return carry

            lax.fori_loop(0, bw // 16, body, 0)

        for k in range(3):
            fire_gather(jnp.int32(k), k)

        def block(b, carry):
            s_base = b * 6
            for j in range(6):
                s = s_base + j
                wait_gather(j)

                @pl.when(s >= 2)
                def _():
                    drain_out(s - 2, j)

                compact(j)

                @pl.when(s + 3 < S)
                def _():
                    fire_gather(s + 3, j)

                fire_out(s, j)
            return carry

        lax.fori_loop(0, n_blocks, block, 0)

        for j, s in ((0, S - 2), (1, S - 1)):
            wait_gather(j)
            drain_out(s - 2, j)
            compact(j)
            fire_out(s, j)
        drain_out(S - 2, 0)
        drain_out(S - 1, 1)

    return emb


def kernel(x, weight):
    Bb, S = x.shape
    V, D = weight.shape
    S_pad = (S + 7) // 8 * 8
    xT = jnp.pad(x.T, ((0, S_pad - S), (0, 0)))
    emb = _build_emb(S, Bb, V, D, num_cores=2, num_subcores=16)
    out5 = emb(xT, weight)
    outT = out5.transpose(0, 1, 3, 2, 4).reshape(S, D, Bb)
    return outT.transpose(2, 0, 1)


# speedup vs baseline: 2.1964x; 1.5109x over previous
import functools

import jax
import jax.numpy as jnp
from jax import lax
from jax.experimental import pallas as pl
from jax.experimental.pallas import tpu as pltpu
from jax.experimental.pallas import tpu_sc as plsc

EMBEDDING_DIM = 32


def _build_emb(S, Bb, V, D, num_cores, num_subcores):
    nw = num_cores * num_subcores
    bw = Bb // nw
    ntb = bw // 128
    ntd = D // 8
    S_pad = (S + 7) // 8 * 8
    assert S == 50 and (S - 2) % 6 == 0
    n_blocks = (S - 2) // 6
    mesh = plsc.VectorSubcoreMesh(core_axis_name="c", subcore_axis_name="s")

    @functools.partial(
        pl.kernel,
        mesh=mesh,
        out_type=jax.ShapeDtypeStruct((S, ntd, Bb // 128, 8, 128), jnp.float32),
        scratch_types=[
            pltpu.VMEM((3, bw), jnp.int32),
            pltpu.VMEM((3, bw, D), jnp.float32),
            pltpu.VMEM((2, D, bw), jnp.float32),
            pltpu.SemaphoreType.DMA,
            pltpu.SemaphoreType.DMA,
            pltpu.SemaphoreType.DMA,
            pltpu.SemaphoreType.DMA,
            pltpu.SemaphoreType.DMA,
        ],
        compiler_params=pltpu.CompilerParams(
            use_tc_tiling_on_sc=False, needs_layout_passes=False),
    )
    def emb(xT, w_rows, out5, idx_v, stage, outstage, gs0, gs1, gs2, os0, os1):
        gsems = [gs0, gs1, gs2]
        osems = [os0, os1]
        wid = lax.axis_index("s") * num_cores + lax.axis_index("c")
        b0 = wid * bw
        tb0 = wid * ntb
        iota16 = lax.iota(jnp.int32, 16)

        def fire_gather(s, ph):
            g = ph % 3
            pltpu.sync_copy(xT.at[s, pl.ds(b0, bw)], idx_v.at[g])
            pltpu.async_copy(w_rows.at[idx_v.at[g]], stage.at[g], gsems[g])

        def wait_gather(ph):
            g = ph % 3
            pltpu.make_async_copy(
                w_rows.at[idx_v.at[g]], stage.at[g], gsems[g]).wait()

        cvs = [lax.rem(iota16 + d0, 16) + 16 * h
               for h in range(D // 16) for d0 in range(16)]

        def fire_out(s, ph):
            o = ph % 2
            for td in range(ntd):
                for tbl in range(ntb):
                    pltpu.async_copy(
                        outstage.at[o, pl.ds(8 * td, 8), pl.ds(128 * tbl, 128)],
                        out5.at[s, td, tb0 + tbl],
                        osems[o])

        def drain_out(s_old, ph):
            o = ph % 2
            for td in range(ntd):
                for tbl in range(ntb):
                    pltpu.make_async_copy(
                        outstage.at[o, pl.ds(8 * td, 8), pl.ds(128 * tbl, 128)],
                        out5.at[s_old, td, tb0 + tbl],
                        osems[o]).wait()

        def compact(ph):
            g, o = ph % 3, ph % 2

            def body(r, carry):
                rv = iota16 + r * 16
                for cv in cvs:
                    vals = plsc.load_gather(stage.at[g], [rv, cv])
                    plsc.store_scatter(outstage.at[o], [cv, rv], vals)
                return carry

            lax.fori_loop(0, bw // 16, body, 0)

        for k in range(3):
            fire_gather(jnp.int32(k), k)

        def block(b, carry):
            s_base = b * 6
            for j in range(6):
                s = s_base + j
                wait_gather(j)

                @pl.when(s >= 2)
                def _():
                    drain_out(s - 2, j)

                compact(j)

                @pl.when(s + 3 < S)
                def _():
                    fire_gather(s + 3, j)

                fire_out(s, j)
            return carry

        lax.fori_loop(0, n_blocks, block, 0)

        for j, s in ((0, S - 2), (1, S - 1)):
            wait_gather(j)
            drain_out(s - 2, j)
            compact(j)
            fire_out(s, j)
        drain_out(S - 2, 0)
        drain_out(S - 1, 1)

    return emb


def kernel(x, weight):
    Bb, S = x.shape
    V, D = weight.shape
    S_pad = (S + 7) // 8 * 8
    xT = jnp.pad(x.T, ((0, S_pad - S), (0, 0)))
    emb = _build_emb(S, Bb, V, D, num_cores=2, num_subcores=16)
    out5 = emb(xT, weight)
    outT = out5.transpose(0, 1, 3, 2, 4).reshape(S, D, Bb)
    return outT.transpose(2, 0, 1)


# speedup vs baseline: 2.2789x; 1.0376x over previous
import functools

import jax
import jax.numpy as jnp
from jax import lax
from jax.experimental import pallas as pl
from jax.experimental.pallas import tpu as pltpu
from jax.experimental.pallas import tpu_sc as plsc

EMBEDDING_DIM = 32


def _build_emb(S, Bb, V, D, num_cores, num_subcores):
    nw = num_cores * num_subcores
    bw = Bb // nw
    ntb = bw // 128
    ntd = D // 8
    S_pad = (S + 7) // 8 * 8
    assert S == 50 and (S - 2) % 6 == 0
    n_blocks = (S - 2) // 6
    mesh = plsc.VectorSubcoreMesh(core_axis_name="c", subcore_axis_name="s")

    @functools.partial(
        pl.kernel,
        mesh=mesh,
        out_type=jax.ShapeDtypeStruct((S, ntd, Bb // 128, 8, 128), jnp.float32),
        scratch_types=[
            pltpu.VMEM((6, bw), jnp.int32),
            pltpu.VMEM((3, bw, D), jnp.float32),
            pltpu.VMEM((2, D, bw), jnp.float32),
            [pltpu.SemaphoreType.DMA] * 6,
            [pltpu.SemaphoreType.DMA] * 3,
            [pltpu.SemaphoreType.DMA] * 2,
        ],
        compiler_params=pltpu.CompilerParams(
            use_tc_tiling_on_sc=False, needs_layout_passes=False),
    )
    def emb(xT, w_rows, out5, idx_v, stage, outstage, isems, gsems, osems):
        wid = lax.axis_index("s") * num_cores + lax.axis_index("c")
        b0 = wid * bw
        tb0 = wid * ntb
        iota16 = lax.iota(jnp.int32, 16)
        cvs = [lax.rem(iota16 + d0, 16) + 16 * h
               for h in range(D // 16) for d0 in range(16)]

        def fire_idx(s, ph):
            i = ph % 6
            pltpu.async_copy(xT.at[s, pl.ds(b0, bw)], idx_v.at[i], isems[i])

        def wait_idx(s, ph):
            i = ph % 6
            pltpu.make_async_copy(
                xT.at[s, pl.ds(b0, bw)], idx_v.at[i], isems[i]).wait()

        def fire_gather(ph):
            i, g = ph % 6, ph % 3
            pltpu.async_copy(w_rows.at[idx_v.at[i]], stage.at[g], gsems[g])

        def wait_gather(ph):
            i, g = ph % 6, ph % 3
            pltpu.make_async_copy(
                w_rows.at[idx_v.at[i]], stage.at[g], gsems[g]).wait()

        def fire_out(s, ph):
            o = ph % 2
            for td in range(ntd):
                for tbl in range(ntb):
                    pltpu.async_copy(
                        outstage.at[o, pl.ds(8 * td, 8), pl.ds(128 * tbl, 128)],
                        out5.at[s, td, tb0 + tbl],
                        osems[o])

        def drain_out(s_old, ph):
            o = ph % 2
            for td in range(ntd):
                for tbl in range(ntb):
                    pltpu.make_async_copy(
                        outstage.at[o, pl.ds(8 * td, 8), pl.ds(128 * tbl, 128)],
                        out5.at[s_old, td, tb0 + tbl],
                        osems[o]).wait()

        def compact(ph):
            g, o = ph % 3, ph % 2

            def body(r, carry):
                rv = iota16 + r * 16
                for cv in cvs:
                    vals = plsc.load_gather(stage.at[g], [rv, cv])
                    plsc.store_scatter(outstage.at[o], [cv, rv], vals)
                return carry

            lax.fori_loop(0, bw // 16, body, 0)

        for k in range(5):
            fire_idx(jnp.int32(k), k)
        for k in range(3):
            wait_idx(jnp.int32(k), k)
            fire_gather(k)

        def block(b, carry):
            s_base = b * 6
            for j in range(6):
                s = s_base + j
                wait_gather(j)

                @pl.when(s >= 2)
                def _():
                    drain_out(s - 2, j)

                compact(j)

                @pl.when(s + 5 < S)
                def _():
                    fire_idx(s + 5, j + 5)

                @pl.when(s + 3 < S)
                def _():
                    wait_idx(s + 3, j + 3)
                    fire_gather(j + 3)

                fire_out(s, j)
            return carry

        lax.fori_loop(0, n_blocks, block, 0)

        for j, s in ((0, S - 2), (1, S - 1)):
            wait_gather(j)
            drain_out(s - 2, j)
            compact(j)
            fire_out(s, j)
        drain_out(S - 2, 0)
        drain_out(S - 1, 1)

    return emb


def kernel(x, weight):
    Bb, S = x.shape
    V, D = weight.shape
    S_pad = (S + 7) // 8 * 8
    xT = jnp.pad(x.T, ((0, S_pad - S), (0, 0)))
    emb = _build_emb(S, Bb, V, D, num_cores=2, num_subcores=16)
    out5 = emb(xT, weight)
    outT = out5.transpose(0, 1, 3, 2, 4).reshape(S, D, Bb)
    return outT.transpose(2, 0, 1)


# speedup vs baseline: 2.3592x; 1.0352x over previous
import functools

import jax
import jax.numpy as jnp
from jax import lax
from jax.experimental import pallas as pl
from jax.experimental.pallas import tpu as pltpu
from jax.experimental.pallas import tpu_sc as plsc

EMBEDDING_DIM = 32


def _build_emb(S, Bb, V, D, num_cores, num_subcores):
    nw = num_cores * num_subcores
    bw = Bb // nw
    ntb = bw // 128
    ntd = D // 8
    S_pad = (S + 7) // 8 * 8
    assert S == 50 and (S - 2) % 6 == 0
    n_blocks = (S - 2) // 6
    mesh = plsc.VectorSubcoreMesh(core_axis_name="c", subcore_axis_name="s")

    @functools.partial(
        pl.kernel,
        mesh=mesh,
        out_type=jax.ShapeDtypeStruct((S, ntd, Bb // 128, 8, 128), jnp.float32),
        scratch_types=[
            pltpu.VMEM((6, bw), jnp.int32),
            pltpu.VMEM((3, bw, D), jnp.float32),
            pltpu.VMEM((2, ntd, ntb, 8, 128), jnp.float32),
            [pltpu.SemaphoreType.DMA] * 6,
            [pltpu.SemaphoreType.DMA] * 3,
            [pltpu.SemaphoreType.DMA] * 2,
        ],
        compiler_params=pltpu.CompilerParams(
            use_tc_tiling_on_sc=False, needs_layout_passes=False),
    )
    def emb(xT, w_rows, out5, idx_v, stage, outstage, isems, gsems, osems):
        wid = lax.axis_index("s") * num_cores + lax.axis_index("c")
        b0 = wid * bw
        tb0 = wid * ntb
        iota16 = lax.iota(jnp.int32, 16)
        cvs = [lax.rem(iota16 + d0, 16) + 16 * h
               for h in range(D // 16) for d0 in range(16)]
        cvs = [(cv, lax.shift_right_logical(cv, 3), lax.bitwise_and(cv, 7))
               for cv in cvs]

        def fire_idx(s, ph):
            i = ph % 6
            pltpu.async_copy(xT.at[s, pl.ds(b0, bw)], idx_v.at[i], isems[i])

        def wait_idx(s, ph):
            i = ph % 6
            pltpu.make_async_copy(
                xT.at[s, pl.ds(b0, bw)], idx_v.at[i], isems[i]).wait()

        def fire_gather(ph):
            i, g = ph % 6, ph % 3
            pltpu.async_copy(w_rows.at[idx_v.at[i]], stage.at[g], gsems[g])

        def wait_gather(ph):
            i, g = ph % 6, ph % 3
            pltpu.make_async_copy(
                w_rows.at[idx_v.at[i]], stage.at[g], gsems[g]).wait()

        def fire_out(s, ph):
            o = ph % 2
            for td in range(ntd):
                pltpu.async_copy(
                    outstage.at[o, td], out5.at[s, td, pl.ds(tb0, ntb)],
                    osems[o])

        def drain_out(s_old, ph):
            o = ph % 2
            for td in range(ntd):
                pltpu.make_async_copy(
                    outstage.at[o, td], out5.at[s_old, td, pl.ds(tb0, ntb)],
                    osems[o]).wait()

        def compact(ph):
            g, o = ph % 3, ph % 2

            def body(r, carry):
                rv = iota16 + r * 16
                tbv = lax.shift_right_logical(rv, 7)
                blv = lax.bitwise_and(rv, 127)
                for cv, tdv, dsv in cvs:
                    vals = plsc.load_gather(stage.at[g], [rv, cv])
                    plsc.store_scatter(
                        outstage.at[o], [tdv, tbv, dsv, blv], vals)
                return carry

            lax.fori_loop(0, bw // 16, body, 0)

        for k in range(5):
            fire_idx(jnp.int32(k), k)
        for k in range(3):
            wait_idx(jnp.int32(k), k)
            fire_gather(k)

        def block(b, carry):
            s_base = b * 6
            for j in range(6):
                s = s_base + j
                wait_gather(j)

                @pl.when(s >= 2)
                def _():
                    drain_out(s - 2, j)

                compact(j)

                @pl.when(s + 5 < S)
                def _():
                    fire_idx(s + 5, j + 5)

                @pl.when(s + 3 < S)
                def _():
                    wait_idx(s + 3, j + 3)
                    fire_gather(j + 3)

                fire_out(s, j)
            return carry

        lax.fori_loop(0, n_blocks, block, 0)

        for j, s in ((0, S - 2), (1, S - 1)):
            wait_gather(j)
            drain_out(s - 2, j)
            compact(j)
            fire_out(s, j)
        drain_out(S - 2, 0)
        drain_out(S - 1, 1)

    return emb


def kernel(x, weight):
    Bb, S = x.shape
    V, D = weight.shape
    S_pad = (S + 7) // 8 * 8
    xT = jnp.pad(x.T, ((0, S_pad - S), (0, 0)))
    emb = _build_emb(S, Bb, V, D, num_cores=2, num_subcores=16)
    out5 = emb(xT, weight)
    outT = out5.transpose(0, 1, 3, 2, 4).reshape(S, D, Bb)
    return outT.transpose(2, 0, 1)
